# baseline (speedup 1.0000x reference)
import functools
import math

import jax
import jax.numpy as jnp
from jax import lax
from jax.experimental import pallas as pl
from jax.experimental.pallas import tpu as pltpu
from jax.experimental.pallas import tpu_sc as plsc

_B, _S, _D = 4, 2048, 1024
_SCALE = math.sqrt(_D)
_NW = 32
_RPW = (_B * _S) // _NW
_CH = 32
_NCH = _RPW // _CH
_LANES = 16
_VPR = _D // _LANES


def _embed_kernel(x_hbm, tok_hbm, pos_hbm, out_hbm, idx_v, tok_v, pos_v, sem):
    c = lax.axis_index("c")
    s = lax.axis_index("s")
    wid = s * 2 + c
    base = wid * _RPW
    pos_base = lax.rem(base, _S)

    pltpu.sync_copy(x_hbm.at[pl.ds(base, _RPW)], idx_v)

    for ch in range(_NCH):
        r0 = ch * _CH
        gat = pltpu.async_copy(
            tok_hbm.at[idx_v.at[pl.ds(r0, _CH)]], tok_v, sem)
        pltpu.sync_copy(pos_hbm.at[pl.ds(pos_base + r0, _CH)], pos_v)
        gat.wait()

        def row_body(r, carry):
            def col_body(cc, carry2):
                off = cc * _LANES
                t = tok_v[r, pl.ds(off, _LANES)]
                p = pos_v[r, pl.ds(off, _LANES)]
                tok_v[r, pl.ds(off, _LANES)] = t * _SCALE + p
                return carry2
            return lax.fori_loop(0, _VPR, col_body, carry)
        lax.fori_loop(0, _CH, row_body, 0)

        pltpu.sync_copy(tok_v, out_hbm.at[pl.ds(base + r0, _CH)])


def kernel(x, token_table, pos_table):
    xf = x.reshape(_B * _S).astype(jnp.int32)
    mesh = plsc.VectorSubcoreMesh(core_axis_name="c", subcore_axis_name="s")
    run = pl.kernel(
        _embed_kernel,
        out_type=jax.ShapeDtypeStruct((_B * _S, _D), jnp.float32),
        mesh=mesh,
        scratch_types=[
            pltpu.VMEM((_RPW,), jnp.int32),
            pltpu.VMEM((_CH, _D), jnp.float32),
            pltpu.VMEM((_CH, _D), jnp.float32),
            pltpu.SemaphoreType.DMA,
        ],
    )
    out = run(xf, token_table, pos_table)
    return out.reshape(_B, _S, _D)


# speedup vs baseline: 2.1700x; 2.1700x over previous
import math

import jax
import jax.numpy as jnp
from jax import lax
from jax.experimental import pallas as pl
from jax.experimental.pallas import tpu as pltpu
from jax.experimental.pallas import tpu_sc as plsc

_B, _S, _D = 4, 2048, 1024
_SCALE = math.sqrt(_D)
_NW = 32
_RPW = (_B * _S) // _NW
_CH = 16
_NCH = _RPW // _CH
_LANES = 16
_VPR = _D // _LANES


def _embed_kernel(x_hbm, tok_hbm, pos_hbm, out_hbm, idx_v,
                  tok0, tok1, pos0, pos1,
                  gs0, gs1, ps0, ps1, ss0, ss1):
    toks = (tok0, tok1)
    poss = (pos0, pos1)
    gsems = (gs0, gs1)
    psems = (ps0, ps1)
    ssems = (ss0, ss1)

    c = lax.axis_index("c")
    s = lax.axis_index("s")
    wid = s * 2 + c
    base = wid * _RPW
    pos_base = lax.rem(base, _S)

    pltpu.sync_copy(x_hbm.at[pl.ds(base, _RPW)], idx_v)

    def start_loads(ch):
        b = ch % 2
        g = pltpu.async_copy(
            tok_hbm.at[idx_v.at[pl.ds(ch * _CH, _CH)]], toks[b], gsems[b])
        p = pltpu.async_copy(
            pos_hbm.at[pl.ds(pos_base + ch * _CH, _CH)], poss[b], psems[b])
        return g, p

    loads = [None] * _NCH
    stores = [None] * _NCH
    loads[0] = start_loads(0)
    for ch in range(_NCH):
        b = ch % 2
        if ch + 1 < _NCH:
            if ch >= 1 and stores[ch - 1] is not None:
                stores[ch - 1].wait()
            loads[ch + 1] = start_loads(ch + 1)
        g, p = loads[ch]
        g.wait()
        p.wait()

        def row_body(r, carry):
            for k in range(_VPR):
                t = toks[b][r, pl.ds(k * _LANES, _LANES)]
                pv = poss[b][r, pl.ds(k * _LANES, _LANES)]
                toks[b][r, pl.ds(k * _LANES, _LANES)] = t * _SCALE + pv
            return carry
        lax.fori_loop(0, _CH, row_body, 0, unroll=False)

        stores[ch] = pltpu.async_copy(
            toks[b], out_hbm.at[pl.ds(base + ch * _CH, _CH)], ssems[b])
    stores[_NCH - 2].wait()
    stores[_NCH - 1].wait()


def kernel(x, token_table, pos_table):
    xf = x.reshape(_B * _S).astype(jnp.int32)
    mesh = plsc.VectorSubcoreMesh(core_axis_name="c", subcore_axis_name="s")
    run = pl.kernel(
        _embed_kernel,
        out_type=jax.ShapeDtypeStruct((_B * _S, _D), jnp.float32),
        mesh=mesh,
        scratch_types=[
            pltpu.VMEM((_RPW,), jnp.int32),
            pltpu.VMEM((_CH, _D), jnp.float32),
            pltpu.VMEM((_CH, _D), jnp.float32),
            pltpu.VMEM((_CH, _D), jnp.float32),
            pltpu.VMEM((_CH, _D), jnp.float32),
            pltpu.SemaphoreType.DMA,
            pltpu.SemaphoreType.DMA,
            pltpu.SemaphoreType.DMA,
            pltpu.SemaphoreType.DMA,
            pltpu.SemaphoreType.DMA,
            pltpu.SemaphoreType.DMA,
        ],
    )
    out = run(xf, token_table, pos_table)
    return out.reshape(_B, _S, _D)
